# baseline (device time: 21459 ns/iter reference)
import jax
import jax.numpy as jnp
from jax import lax
from jax.experimental import pallas as pl
from jax.experimental.pallas import tpu as pltpu

NCHUNK = 8


def kernel(A, B):
    m, k = A.shape
    _, n = B.shape
    nc = n // NCHUNK

    def body(
        a_hbm, b_hbm, out_hbm,
        a_vmem, b_vmem, stage, qsend, qrecv, ssend, srecv,
        in_sems, out_sems, qsend_sems, qrecv_sems, ssend_sems, srecv_sems,
    ):
        my_x = lax.axis_index("x")
        my_y = lax.axis_index("y")
        peer = (my_x, 1 - my_y)

        a_cp = pltpu.make_async_copy(a_hbm, a_vmem, in_sems.at[0])
        b_cp = pltpu.make_async_copy(b_hbm, b_vmem, in_sems.at[1])
        a_cp.start()
        b_cp.start()

        barrier_sem = pltpu.get_barrier_semaphore()
        pl.semaphore_signal(
            barrier_sem, inc=1,
            device_id=peer, device_id_type=pl.DeviceIdType.MESH,
        )
        pl.semaphore_wait(barrier_sem, 1)

        a_cp.wait()
        b_cp.wait()
        a = a_vmem[:, :].astype(jnp.bfloat16)

        rdmas = []
        for c in range(NCHUNK):
            b = b_vmem[:, pl.ds(c * nc, nc)].astype(jnp.bfloat16)
            partial = jnp.dot(a, b, preferred_element_type=jnp.float32)
            stage[c, :, :] = partial.astype(jnp.bfloat16)

            amax = jnp.maximum(jnp.max(jnp.abs(partial)), 1e-20)
            qsend[c, :, :] = jnp.round(partial * (127.0 / amax)).astype(jnp.int8)
            ssend[c, :, :] = jnp.full((8, 128), amax / 127.0, jnp.float32)

            qr = pltpu.make_async_remote_copy(
                src_ref=qsend.at[c],
                dst_ref=qrecv.at[c],
                send_sem=qsend_sems.at[c],
                recv_sem=qrecv_sems.at[c],
                device_id=peer,
                device_id_type=pl.DeviceIdType.MESH,
            )
            qr.start()
            sr = pltpu.make_async_remote_copy(
                src_ref=ssend.at[c],
                dst_ref=srecv.at[c],
                send_sem=ssend_sems.at[c],
                recv_sem=srecv_sems.at[c],
                device_id=peer,
                device_id_type=pl.DeviceIdType.MESH,
            )
            sr.start()
            rdmas.append((qr, sr))

        out_cps = []
        for c in range(NCHUNK):
            qr, sr = rdmas[c]
            qr.wait_recv()
            sr.wait_recv()
            deq = qrecv[c].astype(jnp.float32) * srecv[c, 0, 0]
            stage[c, :, :] = (stage[c].astype(jnp.float32) + deq).astype(
                jnp.bfloat16
            )
            ocp = pltpu.make_async_copy(
                stage.at[c], out_hbm.at[:, pl.ds(c * nc, nc)], out_sems.at[c]
            )
            ocp.start()
            out_cps.append(ocp)

        for c in range(NCHUNK):
            qr, sr = rdmas[c]
            qr.wait_send()
            sr.wait_send()
            out_cps[c].wait()

    return pl.pallas_call(
        body,
        out_shape=jax.ShapeDtypeStruct((m, n), jnp.bfloat16),
        in_specs=[
            pl.BlockSpec(memory_space=pl.ANY),
            pl.BlockSpec(memory_space=pl.ANY),
        ],
        out_specs=pl.BlockSpec(memory_space=pl.ANY),
        scratch_shapes=[
            pltpu.VMEM((m, k), jnp.float32),
            pltpu.VMEM((k, n), jnp.float32),
            pltpu.VMEM((NCHUNK, m, nc), jnp.bfloat16),
            pltpu.VMEM((NCHUNK, m, nc), jnp.int8),
            pltpu.VMEM((NCHUNK, m, nc), jnp.int8),
            pltpu.VMEM((NCHUNK, 8, 128), jnp.float32),
            pltpu.VMEM((NCHUNK, 8, 128), jnp.float32),
            pltpu.SemaphoreType.DMA((2,)),
            pltpu.SemaphoreType.DMA((NCHUNK,)),
            pltpu.SemaphoreType.DMA((NCHUNK,)),
            pltpu.SemaphoreType.DMA((NCHUNK,)),
            pltpu.SemaphoreType.DMA((NCHUNK,)),
            pltpu.SemaphoreType.DMA((NCHUNK,)),
        ],
        compiler_params=pltpu.CompilerParams(collective_id=0),
    )(A, B)


# device time: 20944 ns/iter; 1.0246x vs baseline; 1.0246x over previous
import jax
import jax.numpy as jnp
from jax import lax
from jax.experimental import pallas as pl
from jax.experimental.pallas import tpu as pltpu

NCHUNK = 4


def kernel(A, B):
    m, k = A.shape
    _, n = B.shape
    nc = n // NCHUNK

    def body(
        a_ref, b_ref, out_ref,
        local_buf, qsend, qrecv, ssend, srecv,
        qsend_sems, qrecv_sems, ssend_sems, srecv_sems,
    ):
        my_x = lax.axis_index("x")
        my_y = lax.axis_index("y")
        peer = (my_x, 1 - my_y)

        barrier_sem = pltpu.get_barrier_semaphore()
        pl.semaphore_signal(
            barrier_sem, inc=1,
            device_id=peer, device_id_type=pl.DeviceIdType.MESH,
        )
        pl.semaphore_wait(barrier_sem, 1)

        a = a_ref[:, :].astype(jnp.bfloat16)

        rdmas = []
        for c in range(NCHUNK):
            b = b_ref[:, pl.ds(c * nc, nc)].astype(jnp.bfloat16)
            partial = jnp.dot(a, b, preferred_element_type=jnp.float32)
            local_buf[c, :, :] = partial

            amax = jnp.maximum(jnp.max(jnp.abs(partial)), 1e-20)
            qsend[c, :, :] = jnp.round(partial * (127.0 / amax)).astype(jnp.int8)
            ssend[c, :, :] = jnp.full((8, 128), amax / 127.0, jnp.float32)

            qr = pltpu.make_async_remote_copy(
                src_ref=qsend.at[c],
                dst_ref=qrecv.at[c],
                send_sem=qsend_sems.at[c],
                recv_sem=qrecv_sems.at[c],
                device_id=peer,
                device_id_type=pl.DeviceIdType.MESH,
            )
            qr.start()
            sr = pltpu.make_async_remote_copy(
                src_ref=ssend.at[c],
                dst_ref=srecv.at[c],
                send_sem=ssend_sems.at[c],
                recv_sem=srecv_sems.at[c],
                device_id=peer,
                device_id_type=pl.DeviceIdType.MESH,
            )
            sr.start()
            rdmas.append((qr, sr))

        for c in range(NCHUNK):
            qr, sr = rdmas[c]
            qr.wait_recv()
            sr.wait_recv()
            deq = qrecv[c].astype(jnp.float32) * srecv[c, 0, 0]
            out_ref[:, pl.ds(c * nc, nc)] = (
                local_buf[c] + deq
            ).astype(jnp.bfloat16)

        for c in range(NCHUNK):
            qr, sr = rdmas[c]
            qr.wait_send()
            sr.wait_send()

    return pl.pallas_call(
        body,
        out_shape=jax.ShapeDtypeStruct((m, n), jnp.bfloat16),
        in_specs=[
            pl.BlockSpec(memory_space=pltpu.VMEM),
            pl.BlockSpec(memory_space=pltpu.VMEM),
        ],
        out_specs=pl.BlockSpec(memory_space=pltpu.VMEM),
        scratch_shapes=[
            pltpu.VMEM((NCHUNK, m, nc), jnp.float32),
            pltpu.VMEM((NCHUNK, m, nc), jnp.int8),
            pltpu.VMEM((NCHUNK, m, nc), jnp.int8),
            pltpu.VMEM((NCHUNK, 8, 128), jnp.float32),
            pltpu.VMEM((NCHUNK, 8, 128), jnp.float32),
            pltpu.SemaphoreType.DMA((NCHUNK,)),
            pltpu.SemaphoreType.DMA((NCHUNK,)),
            pltpu.SemaphoreType.DMA((NCHUNK,)),
            pltpu.SemaphoreType.DMA((NCHUNK,)),
        ],
        compiler_params=pltpu.CompilerParams(collective_id=0),
    )(A, B)
